# baseline (speedup 1.0000x reference)
import functools
import math

import jax
import jax.numpy as jnp
from jax import lax
from jax.experimental import pallas as pl
from jax.experimental.pallas import tpu as pltpu
from jax.experimental.pallas import tpu_sc as plsc

F = 2048
O = 128
D = 128
NUM_SENSORS = 100
X_MIN = -2.5
X_MAX = 2.5
PNR = 10.0
EXPECTED = 8.0

_BW = (X_MAX - X_MIN) / NUM_SENSORS
_NP = 10.0 ** (-PNR / 10.0)
_INV2BW2 = 1.0 / (2.0 * _BW * _BW)
_INV2NP2 = 1.0 / (2.0 * _NP * _NP)
_REPL = max(1.0, O / EXPECTED)
_CONST = (-math.log(_BW) - 0.5 * math.log(2.0 * math.pi) + math.log(EXPECTED)
          + math.log(X_MAX - X_MIN) - math.log(D - EXPECTED) - math.log(_REPL))

_NW = 32
_FPW = F // _NW
_NBUF = 4
_CH = O // 16


def _sc_body(pos_hbm, obs_hbm, sen_hbm, wb_hbm, out_hbm,
             pos_v, a_v, b_v, p2_v, wb_v, out_v, s0, s1, s2, s3):
    sems = [s0, s1, s2, s3]
    wid = lax.axis_index("s") * 2 + lax.axis_index("c")
    base = wid * _FPW
    pltpu.sync_copy(pos_hbm.at[pl.ds(base, _FPW)], pos_v)
    pltpu.sync_copy(obs_hbm.at[pl.ds(base, _FPW)], b_v)
    pltpu.sync_copy(sen_hbm.at[pl.ds(base, _FPW)], a_v)
    pltpu.sync_copy(wb_hbm, wb_v)
    wbv = wb_v[...]
    w = wbv[0]
    bb = wbv[1]

    def prep(fi, carry):
        for j in range(_CH):
            sl = pl.ds(j * 16, 16)
            sen = a_v[fi, sl]
            obs = b_v[fi, sl]
            pos = pos_v[fi, sl]
            x = (sen - 0.5) * w + bb
            conf = 1.0 / (1.0 + jnp.exp(-x))
            ct = (2.0 * conf - 1.0) * _INV2NP2 + _CONST
            a_v[fi, sl] = ct - _INV2BW2 * obs * obs
            b_v[fi, sl] = (2.0 * _INV2BW2) * obs
            p2_v[fi, sl] = -_INV2BW2 * pos * pos
        return carry

    lax.fori_loop(0, _FPW, prep, 0)

    def compute_frame(fi, slot):
        poss = [pos_v[fi, pl.ds(j * 16, 16)] for j in range(_CH)]
        p2s = [p2_v[fi, pl.ds(j * 16, 16)] for j in range(_CH)]

        @plsc.parallel_loop(0, D // 16)
        def dchunk(c):
            dbase = c * 16
            advec = a_v[fi, pl.ds(dbase, 16)]
            bdvec = b_v[fi, pl.ds(dbase, 16)]
            for di in range(16):
                ad = advec[di]
                bd = bdvec[di]
                for j in range(_CH):
                    out_v[slot, dbase + di, pl.ds(j * 16, 16)] = (
                        bd * poss[j] + (p2s[j] + ad))

    def group(g, carry):
        for bslot in range(_NBUF):
            fi = g * _NBUF + bslot

            @pl.when(g > 0)
            def _wait():
                pltpu.make_async_copy(
                    out_v.at[bslot], out_hbm.at[0], sems[bslot]).wait()

            compute_frame(fi, bslot)
            pltpu.async_copy(out_v.at[bslot], out_hbm.at[base + fi], sems[bslot])
        return carry

    lax.fori_loop(0, _FPW // _NBUF, group, 0)
    for bslot in range(_NBUF):
        pltpu.make_async_copy(out_v.at[bslot], out_hbm.at[0], sems[bslot]).wait()


def _sc_kernel(positions, obs_positions, sensor_raw, wb):
    mesh = plsc.VectorSubcoreMesh(core_axis_name="c", subcore_axis_name="s")
    return pl.kernel(
        _sc_body,
        mesh=mesh,
        out_type=jax.ShapeDtypeStruct((F, D, O), jnp.float32),
        scratch_types=[
            pltpu.VMEM((_FPW, O), jnp.float32),
            pltpu.VMEM((_FPW, D), jnp.float32),
            pltpu.VMEM((_FPW, D), jnp.float32),
            pltpu.VMEM((_FPW, O), jnp.float32),
            pltpu.VMEM((16,), jnp.float32),
            pltpu.VMEM((_NBUF, D, O), jnp.float32),
            pltpu.SemaphoreType.DMA,
            pltpu.SemaphoreType.DMA,
            pltpu.SemaphoreType.DMA,
            pltpu.SemaphoreType.DMA,
        ],
    )(positions, obs_positions, sensor_raw, wb)


def kernel(positions, obs_positions, sensor_raw, W, b):
    wb = jnp.zeros((16,), jnp.float32).at[0].set(W[0, 0]).at[1].set(b[0])
    return _sc_kernel(positions, obs_positions, sensor_raw, wb)


# speedup vs baseline: 1.9635x; 1.9635x over previous
import functools
import math

import jax
import jax.numpy as jnp
from jax import lax
from jax.experimental import pallas as pl
from jax.experimental.pallas import tpu as pltpu
from jax.experimental.pallas import tpu_sc as plsc

F = 2048
O = 128
D = 128
NUM_SENSORS = 100
X_MIN = -2.5
X_MAX = 2.5
PNR = 10.0
EXPECTED = 8.0

_BW = (X_MAX - X_MIN) / NUM_SENSORS
_NP = 10.0 ** (-PNR / 10.0)
_INV2BW2 = 1.0 / (2.0 * _BW * _BW)
_INV2NP2 = 1.0 / (2.0 * _NP * _NP)
_REPL = max(1.0, O / EXPECTED)
_CONST = (-math.log(_BW) - 0.5 * math.log(2.0 * math.pi) + math.log(EXPECTED)
          + math.log(X_MAX - X_MIN) - math.log(D - EXPECTED) - math.log(_REPL))

_GDN = lax.GatherDimensionNumbers(
    offset_dims=(), collapsed_slice_dims=(0,), start_index_map=(0,))

_NW = 32
_FPW = F // _NW
_NBUF = 4
_CH = O // 16


def _sc_body(pos_hbm, obs_hbm, sen_hbm, wb_hbm, out_hbm,
             pos_v, a_v, b_v, p2_v, wb_v, out_v, s0, s1, s2, s3):
    sems = [s0, s1, s2, s3]
    wid = lax.axis_index("s") * 2 + lax.axis_index("c")
    base = wid * _FPW
    pltpu.sync_copy(pos_hbm.at[pl.ds(base, _FPW)], pos_v)
    pltpu.sync_copy(obs_hbm.at[pl.ds(base, _FPW)], b_v)
    pltpu.sync_copy(sen_hbm.at[pl.ds(base, _FPW)], a_v)
    pltpu.sync_copy(wb_hbm, wb_v)
    wbv = wb_v[...]
    w = wbv[0]
    bb = wbv[1]

    def prep(fi, carry):
        for j in range(_CH):
            sl = pl.ds(j * 16, 16)
            sen = a_v[fi, sl]
            obs = b_v[fi, sl]
            pos = pos_v[fi, sl]
            x = (sen - 0.5) * w + bb
            conf = 1.0 / (1.0 + jnp.exp(-x))
            ct = (2.0 * conf - 1.0) * _INV2NP2 + _CONST
            a_v[fi, sl] = ct - _INV2BW2 * obs * obs
            b_v[fi, sl] = (2.0 * _INV2BW2) * obs
            p2_v[fi, sl] = -_INV2BW2 * pos * pos
        return carry

    lax.fori_loop(0, _FPW, prep, 0)

    def compute_frame(fi, slot):
        poss = [pos_v[fi, pl.ds(j * 16, 16)] for j in range(_CH)]
        p2s = [p2_v[fi, pl.ds(j * 16, 16)] for j in range(_CH)]
        def dchunk(c, carry):
            dbase = c * 16
            advec = a_v[fi, pl.ds(dbase, 16)]
            bdvec = b_v[fi, pl.ds(dbase, 16)]
            for di in range(16):
                lane = jnp.full((16, 1), di, jnp.int32)
                adb = lax.gather(advec, lane, _GDN, (1,),
                                 mode=lax.GatherScatterMode.PROMISE_IN_BOUNDS)
                bdb = lax.gather(bdvec, lane, _GDN, (1,),
                                 mode=lax.GatherScatterMode.PROMISE_IN_BOUNDS)
                for j in range(_CH):
                    out_v[slot, dbase + di, pl.ds(j * 16, 16)] = (
                        bdb * poss[j] + (p2s[j] + adb))
            return carry

        lax.fori_loop(0, D // 16, dchunk, 0)

    def group(g, carry):
        for bslot in range(_NBUF):
            fi = g * _NBUF + bslot

            @pl.when(g > 0)
            def _wait():
                pltpu.make_async_copy(
                    out_v.at[bslot], out_hbm.at[0], sems[bslot]).wait()

            compute_frame(fi, bslot)
            pltpu.async_copy(out_v.at[bslot], out_hbm.at[base + fi], sems[bslot])
        return carry

    lax.fori_loop(0, _FPW // _NBUF, group, 0)
    for bslot in range(_NBUF):
        pltpu.make_async_copy(out_v.at[bslot], out_hbm.at[0], sems[bslot]).wait()


def _sc_kernel(positions, obs_positions, sensor_raw, wb):
    mesh = plsc.VectorSubcoreMesh(core_axis_name="c", subcore_axis_name="s")
    return pl.kernel(
        _sc_body,
        mesh=mesh,
        out_type=jax.ShapeDtypeStruct((F, D, O), jnp.float32),
        scratch_types=[
            pltpu.VMEM((_FPW, O), jnp.float32),
            pltpu.VMEM((_FPW, D), jnp.float32),
            pltpu.VMEM((_FPW, D), jnp.float32),
            pltpu.VMEM((_FPW, O), jnp.float32),
            pltpu.VMEM((16,), jnp.float32),
            pltpu.VMEM((_NBUF, D, O), jnp.float32),
            pltpu.SemaphoreType.DMA,
            pltpu.SemaphoreType.DMA,
            pltpu.SemaphoreType.DMA,
            pltpu.SemaphoreType.DMA,
        ],
    )(positions, obs_positions, sensor_raw, wb)


def kernel(positions, obs_positions, sensor_raw, W, b):
    wb = jnp.zeros((16,), jnp.float32).at[0].set(W[0, 0]).at[1].set(b[0])
    return _sc_kernel(positions, obs_positions, sensor_raw, wb)


# speedup vs baseline: 2.0017x; 1.0195x over previous
import functools
import math

import jax
import jax.numpy as jnp
from jax import lax
from jax.experimental import pallas as pl
from jax.experimental.pallas import tpu as pltpu
from jax.experimental.pallas import tpu_sc as plsc

F = 2048
O = 128
D = 128
NUM_SENSORS = 100
X_MIN = -2.5
X_MAX = 2.5
PNR = 10.0
EXPECTED = 8.0

_BW = (X_MAX - X_MIN) / NUM_SENSORS
_NP = 10.0 ** (-PNR / 10.0)
_INV2BW2 = 1.0 / (2.0 * _BW * _BW)
_INV2NP2 = 1.0 / (2.0 * _NP * _NP)
_REPL = max(1.0, O / EXPECTED)
_CONST = (-math.log(_BW) - 0.5 * math.log(2.0 * math.pi) + math.log(EXPECTED)
          + math.log(X_MAX - X_MIN) - math.log(D - EXPECTED) - math.log(_REPL))

_GDN = lax.GatherDimensionNumbers(
    offset_dims=(), collapsed_slice_dims=(0,), start_index_map=(0,))

_NW = 32
_FPW = F // _NW
_NBUF = 4
_CH = O // 16


def _sc_body(pos_hbm, obs_hbm, sen_hbm, wb_hbm, out_hbm,
             pos_v, a_v, b_v, p2_v, wb_v, out_v, s0, s1, s2, s3):
    sems = [s0, s1, s2, s3]
    wid = lax.axis_index("s") * 2 + lax.axis_index("c")
    base = wid * _FPW
    pltpu.async_copy(pos_hbm.at[pl.ds(base, _FPW)], pos_v, s0)
    pltpu.async_copy(obs_hbm.at[pl.ds(base, _FPW)], b_v, s1)
    pltpu.async_copy(sen_hbm.at[pl.ds(base, _FPW)], a_v, s2)
    pltpu.sync_copy(wb_hbm, wb_v)
    pltpu.make_async_copy(pos_hbm.at[pl.ds(base, _FPW)], pos_v, s0).wait()
    pltpu.make_async_copy(obs_hbm.at[pl.ds(base, _FPW)], b_v, s1).wait()
    pltpu.make_async_copy(sen_hbm.at[pl.ds(base, _FPW)], a_v, s2).wait()
    wbv = wb_v[...]
    w = wbv[0]
    bb = wbv[1]

    def prep(fi, carry):
        for j in range(_CH):
            sl = pl.ds(j * 16, 16)
            sen = a_v[fi, sl]
            obs = b_v[fi, sl]
            pos = pos_v[fi, sl]
            x = (sen - 0.5) * w + bb
            conf = 1.0 / (1.0 + jnp.exp(-x))
            ct = (2.0 * conf - 1.0) * _INV2NP2 + _CONST
            a_v[fi, sl] = ct - _INV2BW2 * obs * obs
            b_v[fi, sl] = (2.0 * _INV2BW2) * obs
            p2_v[fi, sl] = -_INV2BW2 * pos * pos
        return carry

    lax.fori_loop(0, _FPW, prep, 0)

    def compute_frame(fi, slot):
        poss = [pos_v[fi, pl.ds(j * 16, 16)] for j in range(_CH)]
        p2s = [p2_v[fi, pl.ds(j * 16, 16)] for j in range(_CH)]
        def dchunk(c, carry):
            dbase = c * 16
            advec = a_v[fi, pl.ds(dbase, 16)]
            bdvec = b_v[fi, pl.ds(dbase, 16)]
            for di in range(16):
                lane = jnp.full((16, 1), di, jnp.int32)
                adb = lax.gather(advec, lane, _GDN, (1,),
                                 mode=lax.GatherScatterMode.PROMISE_IN_BOUNDS)
                bdb = lax.gather(bdvec, lane, _GDN, (1,),
                                 mode=lax.GatherScatterMode.PROMISE_IN_BOUNDS)
                for j in range(_CH):
                    out_v[slot, dbase + di, pl.ds(j * 16, 16)] = (
                        bdb * poss[j] + (p2s[j] + adb))
            return carry

        lax.fori_loop(0, D // 16, dchunk, 0)

    def group(g, carry):
        for bslot in range(_NBUF):
            fi = g * _NBUF + bslot

            @pl.when(g > 0)
            def _wait():
                pltpu.make_async_copy(
                    out_v.at[bslot], out_hbm.at[0], sems[bslot]).wait()

            compute_frame(fi, bslot)
            pltpu.async_copy(out_v.at[bslot], out_hbm.at[base + fi], sems[bslot])
        return carry

    lax.fori_loop(0, _FPW // _NBUF, group, 0)
    for bslot in range(_NBUF):
        pltpu.make_async_copy(out_v.at[bslot], out_hbm.at[0], sems[bslot]).wait()


def _sc_kernel(positions, obs_positions, sensor_raw, wb):
    mesh = plsc.VectorSubcoreMesh(core_axis_name="c", subcore_axis_name="s")
    return pl.kernel(
        _sc_body,
        mesh=mesh,
        out_type=jax.ShapeDtypeStruct((F, D, O), jnp.float32),
        scratch_types=[
            pltpu.VMEM((_FPW, O), jnp.float32),
            pltpu.VMEM((_FPW, D), jnp.float32),
            pltpu.VMEM((_FPW, D), jnp.float32),
            pltpu.VMEM((_FPW, O), jnp.float32),
            pltpu.VMEM((16,), jnp.float32),
            pltpu.VMEM((_NBUF, D, O), jnp.float32),
            pltpu.SemaphoreType.DMA,
            pltpu.SemaphoreType.DMA,
            pltpu.SemaphoreType.DMA,
            pltpu.SemaphoreType.DMA,
        ],
    )(positions, obs_positions, sensor_raw, wb)


def kernel(positions, obs_positions, sensor_raw, W, b):
    wb = jnp.zeros((16,), jnp.float32).at[0].set(W[0, 0]).at[1].set(b[0])
    return _sc_kernel(positions, obs_positions, sensor_raw, wb)
